# baseline (speedup 1.0000x reference)
import functools

import jax
import jax.numpy as jnp
import numpy as np
from jax import lax
from jax.experimental import pallas as pl
from jax.experimental.pallas import tpu as pltpu
from jax.experimental.pallas import tpu_sc as plsc

_NC = 2
_NS = 16
_NW = _NC * _NS
_LANES = 16


def _prep_table(table, Dp):
    V, D = table.shape
    tb = jnp.pad(table.astype(jnp.bfloat16), ((0, 0), (0, Dp - D)))
    w = lax.bitcast_convert_type(tb, jnp.uint16).astype(jnp.uint32)
    packed = w[:, 0::2] | (w[:, 1::2] << 16)
    return lax.bitcast_convert_type(packed, jnp.int32)


def _gather_sum_sc(x, table_b):
    B, S = x.shape
    _, Dpw = table_b.shape
    Dp = 2 * Dpw
    assert B % _NW == 0 and Dp % (2 * _LANES) == 0
    BW = B // _NW
    C0 = 104
    C1 = S - C0
    assert C0 % 8 == 0 and C1 % 8 == 0 and C0 <= 128 and C1 <= 128
    _CHUNKS = ((0, C0), (C0, C1))
    KV = Dp // (2 * _LANES)
    assert S % 2 == 0 and BW % 2 == 0 and BW >= 4

    mesh = plsc.VectorSubcoreMesh(core_axis_name="c", subcore_axis_name="s")

    @functools.partial(
        pl.kernel,
        out_type=jax.ShapeDtypeStruct((B, Dp), jnp.float32),
        mesh=mesh,
        scratch_types=[
            pltpu.VMEM((S,), jnp.int32),
            pltpu.VMEM((S,), jnp.int32),
            pltpu.VMEM((S, Dpw), jnp.int32),
            pltpu.VMEM((S, Dpw), jnp.int32),
            pltpu.VMEM((Dp,), jnp.float32),
            pltpu.VMEM((Dp,), jnp.float32),
            pltpu.SemaphoreType.DMA,
            pltpu.SemaphoreType.DMA,
            pltpu.SemaphoreType.DMA,
        ],
        compiler_params=pltpu.CompilerParams(
            use_tc_tiling_on_sc=False, needs_layout_passes=False),
    )
    def gather_sum(x_hbm, tab_hbm, out_hbm, idx0, idx1, rows0, rows1,
                   st0, st1, gsem, isem, osem):
        wid = lax.axis_index("s") * _NC + lax.axis_index("c")
        base_e = wid * BW

        def fire_gathers(idx_v, rows_v):
            for off, sz in _CHUNKS:
                pltpu.async_copy(
                    tab_hbm.at[idx_v.at[pl.ds(off, sz)]],
                    rows_v.at[pl.ds(off, sz)],
                    gsem,
                )

        def wait_gathers(idx_v, rows_v):
            for off, sz in _CHUNKS:
                pltpu.make_async_copy(
                    tab_hbm.at[idx_v.at[pl.ds(off, sz)]],
                    rows_v.at[pl.ds(off, sz)],
                    gsem,
                ).wait()

        pltpu.sync_copy(x_hbm.at[base_e], idx0)
        fire_gathers(idx0, rows0)
        pltpu.async_copy(x_hbm.at[base_e + 1], idx1, isem)

        bufs = ((idx0, rows0, st0), (idx1, rows1, st1))

        def accumulate(rows_c):
            zero = tuple(
                jnp.zeros((_LANES,), jnp.float32) for _ in range(2 * KV))

            def acc_body(j, acc):
                res = list(acc)
                for p in range(2):
                    for k in range(KV):
                        va = plsc.bitcast(
                            rows_c[j + 2 * p, pl.ds(_LANES * k, _LANES)],
                            jnp.bfloat16)
                        vb = plsc.bitcast(
                            rows_c[j + 2 * p + 1, pl.ds(_LANES * k, _LANES)],
                            jnp.bfloat16)
                        a, b = plsc.unpack(
                            va + vb, format=plsc.PackFormat.INTERLEAVED)
                        res[2 * k] = res[2 * k] + a
                        res[2 * k + 1] = res[2 * k + 1] + b
                return tuple(res)

            return plsc.parallel_loop(0, S, step=4, carry=zero)(acc_body)

        def body(i, carry):
            for b in (0, 1):
                ec = 2 * i + b
                idx_c, rows_c, st_c = bufs[b]
                idx_n, rows_n, _ = bufs[1 - b]

                @pl.when(ec + 1 < BW)
                def _fire():
                    pltpu.make_async_copy(
                        x_hbm.at[base_e], idx_n, isem).wait()
                    fire_gathers(idx_n, rows_n)

                wait_gathers(idx_c, rows_c)

                @pl.when(ec + 2 < BW)
                def _prefetch():
                    pltpu.async_copy(x_hbm.at[base_e + ec + 2], idx_c, isem)

                acc = accumulate(rows_c)

                @pl.when(ec >= 2)
                def _reclaim():
                    pltpu.make_async_copy(
                        st_c, out_hbm.at[base_e], osem).wait()
                for k in range(KV):
                    st_c[pl.ds(2 * _LANES * k, _LANES)] = acc[2 * k]
                    st_c[pl.ds(2 * _LANES * k + _LANES, _LANES)] = (
                        acc[2 * k + 1])
                pltpu.async_copy(st_c, out_hbm.at[base_e + ec], osem)
            return carry

        lax.fori_loop(0, BW // 2, body, 0)
        pltpu.make_async_copy(st0, out_hbm.at[base_e], osem).wait()
        pltpu.make_async_copy(st1, out_hbm.at[base_e], osem).wait()

    return gather_sum(x, table_b)


def _stats_body(s_ref, w1_ref, b1_ref, o_ref, acc_ref, *, seq):
    i = pl.program_id(0)
    h = (
        jnp.dot(s_ref[...], w1_ref[...], preferred_element_type=jnp.float32)
        * (1.0 / seq)
        + b1_ref[...]
    )
    ps = jnp.sum(h, axis=0, keepdims=True)
    ps2 = jnp.sum(h * h, axis=0, keepdims=True)
    st = jnp.concatenate([ps, ps2], axis=0)

    @pl.when(i == 0)
    def _init():
        acc_ref[...] = st

    @pl.when(i > 0)
    def _acc():
        acc_ref[...] = acc_ref[...] + st

    @pl.when(i == pl.num_programs(0) - 1)
    def _out():
        o_ref[...] = acc_ref[...]


def _apply_body(
    s_ref, w1_ref, b1_ref, st_ref, g_ref, be_ref, w2_ref, b2_ref, o_ref,
    *, seq, batch
):
    h = (
        jnp.dot(s_ref[...], w1_ref[...], preferred_element_type=jnp.float32)
        * (1.0 / seq)
        + b1_ref[...]
    )
    mu = st_ref[0:1, :] * (1.0 / batch)
    var = st_ref[1:2, :] * (1.0 / batch) - mu * mu
    inv = lax.rsqrt(var + 1e-5)
    hn = (h - mu) * (g_ref[...] * inv) + be_ref[...]
    hr = jnp.maximum(hn, 0.0)
    o_ref[...] = (
        jnp.dot(hr, w2_ref[...], preferred_element_type=jnp.float32)
        + b2_ref[...]
    )


def kernel(x, table, W1, b1, gamma, beta, W2, b2):
    B, S = x.shape
    V, D = table.shape
    H = W1.shape[1]
    Lb = W2.shape[1]
    Dp = ((D + 2 * _LANES - 1) // (2 * _LANES)) * (2 * _LANES)

    table_b = _prep_table(table, Dp)
    s = _gather_sum_sc(x, table_b)

    q = np.arange(Dp)
    k32, t = q // 32, q % 32
    perm = np.where(t < 16, 32 * k32 + 2 * t, 32 * k32 + 2 * (t - 16) + 1)
    W1p = jnp.pad(W1, ((0, Dp - D), (0, 0)))[perm, :]
    b1r = b1.reshape(1, H)

    BT = 2048
    grid = (B // BT,)
    sums = pl.pallas_call(
        functools.partial(_stats_body, seq=S),
        grid=grid,
        in_specs=[
            pl.BlockSpec((BT, Dp), lambda i: (i, 0)),
            pl.BlockSpec((Dp, H), lambda i: (0, 0)),
            pl.BlockSpec((1, H), lambda i: (0, 0)),
        ],
        out_specs=pl.BlockSpec((2, H), lambda i: (0, 0)),
        out_shape=jax.ShapeDtypeStruct((2, H), jnp.float32),
        scratch_shapes=[pltpu.VMEM((2, H), jnp.float32)],
    )(s, W1p, b1r)

    out = pl.pallas_call(
        functools.partial(_apply_body, seq=S, batch=B),
        grid=grid,
        in_specs=[
            pl.BlockSpec((BT, Dp), lambda i: (i, 0)),
            pl.BlockSpec((Dp, H), lambda i: (0, 0)),
            pl.BlockSpec((1, H), lambda i: (0, 0)),
            pl.BlockSpec((2, H), lambda i: (0, 0)),
            pl.BlockSpec((1, H), lambda i: (0, 0)),
            pl.BlockSpec((1, H), lambda i: (0, 0)),
            pl.BlockSpec((H, Lb), lambda i: (0, 0)),
            pl.BlockSpec((1, Lb), lambda i: (0, 0)),
        ],
        out_specs=pl.BlockSpec((BT, Lb), lambda i: (i, 0)),
        out_shape=jax.ShapeDtypeStruct((B, Lb), jnp.float32),
    )(s, W1p, b1r, sums, gamma.reshape(1, H), beta.reshape(1, H), W2,
      b2.reshape(1, Lb))
    return out


# speedup vs baseline: 3.5828x; 3.5828x over previous
import functools

import jax
import jax.numpy as jnp
import numpy as np
from jax import lax
from jax.experimental import pallas as pl
from jax.experimental.pallas import tpu as pltpu
from jax.experimental.pallas import tpu_sc as plsc

_NC = 2
_NS = 16
_NW = _NC * _NS
_LANES = 16


def _prep_body(t_ref, o_ref, *, pad):
    vb = t_ref.shape[0]
    o_ref[...] = jnp.concatenate(
        [t_ref[...].astype(jnp.bfloat16),
         jnp.zeros((vb, pad), jnp.bfloat16)],
        axis=1,
    )


def _prep_table(table, Dp):
    V, D = table.shape
    VB = 2048
    grid = ((V + VB - 1) // VB,)
    return pl.pallas_call(
        functools.partial(_prep_body, pad=Dp - D),
        grid=grid,
        in_specs=[pl.BlockSpec((VB, D), lambda i: (i, 0))],
        out_specs=pl.BlockSpec((VB, Dp), lambda i: (i, 0)),
        out_shape=jax.ShapeDtypeStruct((V, Dp), jnp.bfloat16),
    )(table)


def _gather_sum_sc(x, table_b):
    B, S = x.shape
    _, Dp = table_b.shape
    assert B % _NW == 0 and Dp % (2 * _LANES) == 0
    BW = B // _NW
    C0 = 104
    C1 = S - C0
    assert C0 % 8 == 0 and C1 % 8 == 0 and C0 <= 128 and C1 <= 128
    _CHUNKS = ((0, C0), (C0, C1))
    KV = Dp // (2 * _LANES)
    assert S % 2 == 0 and BW % 2 == 0 and BW >= 4

    mesh = plsc.VectorSubcoreMesh(core_axis_name="c", subcore_axis_name="s")

    @functools.partial(
        pl.kernel,
        out_type=jax.ShapeDtypeStruct((B, Dp), jnp.float32),
        mesh=mesh,
        scratch_types=[
            pltpu.VMEM((S,), jnp.int32),
            pltpu.VMEM((S,), jnp.int32),
            pltpu.VMEM((S, Dp), jnp.bfloat16),
            pltpu.VMEM((S, Dp), jnp.bfloat16),
            pltpu.VMEM((Dp,), jnp.float32),
            pltpu.VMEM((Dp,), jnp.float32),
            pltpu.SemaphoreType.DMA,
            pltpu.SemaphoreType.DMA,
            pltpu.SemaphoreType.DMA,
        ],
        compiler_params=pltpu.CompilerParams(
            use_tc_tiling_on_sc=False, needs_layout_passes=False),
    )
    def gather_sum(x_hbm, tab_hbm, out_hbm, idx0, idx1, rows0, rows1,
                   st0, st1, gsem, isem, osem):
        wid = lax.axis_index("s") * _NC + lax.axis_index("c")
        base_e = wid * BW

        def fire_gathers(idx_v, rows_v):
            for off, sz in _CHUNKS:
                pltpu.async_copy(
                    tab_hbm.at[idx_v.at[pl.ds(off, sz)]],
                    rows_v.at[pl.ds(off, sz)],
                    gsem,
                )

        def wait_gathers(idx_v, rows_v):
            for off, sz in _CHUNKS:
                pltpu.make_async_copy(
                    tab_hbm.at[idx_v.at[pl.ds(off, sz)]],
                    rows_v.at[pl.ds(off, sz)],
                    gsem,
                ).wait()

        pltpu.sync_copy(x_hbm.at[base_e], idx0)
        fire_gathers(idx0, rows0)
        pltpu.async_copy(x_hbm.at[base_e + 1], idx1, isem)

        bufs = ((idx0, rows0, st0), (idx1, rows1, st1))

        def accumulate(rows_c):
            zero = tuple(
                jnp.zeros((_LANES,), jnp.float32) for _ in range(2 * KV))

            def acc_body(j, acc):
                res = list(acc)
                for p in range(4):
                    for k in range(KV):
                        va = rows_c[j + 2 * p,
                                    pl.ds(2 * _LANES * k, 2 * _LANES)]
                        vb = rows_c[j + 2 * p + 1,
                                    pl.ds(2 * _LANES * k, 2 * _LANES)]
                        a, b = plsc.unpack(
                            va + vb, format=plsc.PackFormat.INTERLEAVED)
                        res[2 * k] = res[2 * k] + a
                        res[2 * k + 1] = res[2 * k + 1] + b
                return tuple(res)

            return plsc.parallel_loop(0, S, step=8, carry=zero)(acc_body)

        def body(i, carry):
            for b in (0, 1):
                ec = 2 * i + b
                idx_c, rows_c, st_c = bufs[b]
                idx_n, rows_n, _ = bufs[1 - b]

                @pl.when(ec + 1 < BW)
                def _fire():
                    pltpu.make_async_copy(
                        x_hbm.at[base_e], idx_n, isem).wait()
                    fire_gathers(idx_n, rows_n)

                wait_gathers(idx_c, rows_c)

                @pl.when(ec + 2 < BW)
                def _prefetch():
                    pltpu.async_copy(x_hbm.at[base_e + ec + 2], idx_c, isem)

                acc = accumulate(rows_c)

                @pl.when(ec >= 2)
                def _reclaim():
                    pltpu.make_async_copy(
                        st_c, out_hbm.at[base_e], osem).wait()
                for k in range(KV):
                    st_c[pl.ds(2 * _LANES * k, _LANES)] = acc[2 * k]
                    st_c[pl.ds(2 * _LANES * k + _LANES, _LANES)] = (
                        acc[2 * k + 1])
                pltpu.async_copy(st_c, out_hbm.at[base_e + ec], osem)
            return carry

        lax.fori_loop(0, BW // 2, body, 0)
        pltpu.make_async_copy(st0, out_hbm.at[base_e], osem).wait()
        pltpu.make_async_copy(st1, out_hbm.at[base_e], osem).wait()

    return gather_sum(x, table_b)


def _stats_body(s_ref, w1_ref, b1_ref, o_ref, acc_ref, *, seq):
    i = pl.program_id(0)
    h = (
        jnp.dot(s_ref[...], w1_ref[...], preferred_element_type=jnp.float32)
        * (1.0 / seq)
        + b1_ref[...]
    )
    ps = jnp.sum(h, axis=0, keepdims=True)
    ps2 = jnp.sum(h * h, axis=0, keepdims=True)
    st = jnp.concatenate([ps, ps2], axis=0)

    @pl.when(i == 0)
    def _init():
        acc_ref[...] = st

    @pl.when(i > 0)
    def _acc():
        acc_ref[...] = acc_ref[...] + st

    @pl.when(i == pl.num_programs(0) - 1)
    def _out():
        o_ref[...] = acc_ref[...]


def _apply_body(
    s_ref, w1_ref, b1_ref, st_ref, g_ref, be_ref, w2_ref, b2_ref, o_ref,
    *, seq, batch
):
    h = (
        jnp.dot(s_ref[...], w1_ref[...], preferred_element_type=jnp.float32)
        * (1.0 / seq)
        + b1_ref[...]
    )
    mu = st_ref[0:1, :] * (1.0 / batch)
    var = st_ref[1:2, :] * (1.0 / batch) - mu * mu
    inv = lax.rsqrt(var + 1e-5)
    hn = (h - mu) * (g_ref[...] * inv) + be_ref[...]
    hr = jnp.maximum(hn, 0.0)
    o_ref[...] = (
        jnp.dot(hr, w2_ref[...], preferred_element_type=jnp.float32)
        + b2_ref[...]
    )


def kernel(x, table, W1, b1, gamma, beta, W2, b2):
    B, S = x.shape
    V, D = table.shape
    H = W1.shape[1]
    Lb = W2.shape[1]
    Dp = ((D + 2 * _LANES - 1) // (2 * _LANES)) * (2 * _LANES)

    table_b = _prep_table(table, Dp)
    s = _gather_sum_sc(x, table_b)

    q = np.arange(Dp)
    k32, t = q // 32, q % 32
    perm = np.where(t < 16, 32 * k32 + 2 * t, 32 * k32 + 2 * (t - 16) + 1)
    W1p = jnp.pad(W1, ((0, Dp - D), (0, 0)))[perm, :]
    b1r = b1.reshape(1, H)

    BT = 2048
    grid = (B // BT,)
    sums = pl.pallas_call(
        functools.partial(_stats_body, seq=S),
        grid=grid,
        in_specs=[
            pl.BlockSpec((BT, Dp), lambda i: (i, 0)),
            pl.BlockSpec((Dp, H), lambda i: (0, 0)),
            pl.BlockSpec((1, H), lambda i: (0, 0)),
        ],
        out_specs=pl.BlockSpec((2, H), lambda i: (0, 0)),
        out_shape=jax.ShapeDtypeStruct((2, H), jnp.float32),
        scratch_shapes=[pltpu.VMEM((2, H), jnp.float32)],
    )(s, W1p, b1r)

    out = pl.pallas_call(
        functools.partial(_apply_body, seq=S, batch=B),
        grid=grid,
        in_specs=[
            pl.BlockSpec((BT, Dp), lambda i: (i, 0)),
            pl.BlockSpec((Dp, H), lambda i: (0, 0)),
            pl.BlockSpec((1, H), lambda i: (0, 0)),
            pl.BlockSpec((2, H), lambda i: (0, 0)),
            pl.BlockSpec((1, H), lambda i: (0, 0)),
            pl.BlockSpec((1, H), lambda i: (0, 0)),
            pl.BlockSpec((H, Lb), lambda i: (0, 0)),
            pl.BlockSpec((1, Lb), lambda i: (0, 0)),
        ],
        out_specs=pl.BlockSpec((BT, Lb), lambda i: (i, 0)),
        out_shape=jax.ShapeDtypeStruct((B, Lb), jnp.float32),
    )(s, W1p, b1r, sums, gamma.reshape(1, H), beta.reshape(1, H), W2,
      b2.reshape(1, Lb))
    return out


# speedup vs baseline: 3.9070x; 1.0905x over previous
import functools

import jax
import jax.numpy as jnp
import numpy as np
from jax import lax
from jax.experimental import pallas as pl
from jax.experimental.pallas import tpu as pltpu
from jax.experimental.pallas import tpu_sc as plsc

_NC = 2
_NS = 16
_NW = _NC * _NS
_LANES = 16


def _prep_body(t_ref, o_ref, *, pad):
    vb = t_ref.shape[0]
    o_ref[...] = jnp.concatenate(
        [t_ref[...].astype(jnp.bfloat16),
         jnp.zeros((vb, pad), jnp.bfloat16)],
        axis=1,
    )


def _prep_table(table, Dp):
    V, D = table.shape
    Vp = ((V + 7) // 8) * 8
    VB = 2048
    grid = ((Vp + VB - 1) // VB,)
    return pl.pallas_call(
        functools.partial(_prep_body, pad=Dp - D),
        grid=grid,
        in_specs=[pl.BlockSpec((VB, D), lambda i: (i, 0))],
        out_specs=pl.BlockSpec((VB, Dp), lambda i: (i, 0)),
        out_shape=jax.ShapeDtypeStruct((Vp, Dp), jnp.bfloat16),
    )(table)


def _gather_sum_sc(x, table_b):
    B, S = x.shape
    _, Dp = table_b.shape
    assert B % _NW == 0 and Dp % (2 * _LANES) == 0
    BW = B // _NW
    C0 = 104
    C1 = S - C0
    assert C0 % 8 == 0 and C1 % 8 == 0 and C0 <= 128 and C1 <= 128
    _CHUNKS = ((0, C0), (C0, C1))
    KV = Dp // (2 * _LANES)
    assert S % 2 == 0 and BW % 2 == 0 and BW >= 4

    mesh = plsc.VectorSubcoreMesh(core_axis_name="c", subcore_axis_name="s")

    @functools.partial(
        pl.kernel,
        out_type=jax.ShapeDtypeStruct((B, Dp), jnp.float32),
        mesh=mesh,
        scratch_types=[
            pltpu.VMEM((S,), jnp.int32),
            pltpu.VMEM((S,), jnp.int32),
            pltpu.VMEM((S, Dp), jnp.bfloat16),
            pltpu.VMEM((S, Dp), jnp.bfloat16),
            pltpu.VMEM((Dp,), jnp.float32),
            pltpu.VMEM((Dp,), jnp.float32),
            pltpu.SemaphoreType.DMA,
            pltpu.SemaphoreType.DMA,
            pltpu.SemaphoreType.DMA,
        ],
        compiler_params=pltpu.CompilerParams(
            use_tc_tiling_on_sc=False, needs_layout_passes=False),
    )
    def gather_sum(x_hbm, tab_hbm, out_hbm, idx0, idx1, rows0, rows1,
                   st0, st1, gsem, isem, osem):
        wid = lax.axis_index("s") * _NC + lax.axis_index("c")
        base_e = wid * BW

        def fire_gathers(idx_v, rows_v):
            for off, sz in _CHUNKS:
                pltpu.async_copy(
                    tab_hbm.at[idx_v.at[pl.ds(off, sz)]],
                    rows_v.at[pl.ds(off, sz)],
                    gsem,
                )

        def wait_gathers(idx_v, rows_v):
            for off, sz in _CHUNKS:
                pltpu.make_async_copy(
                    tab_hbm.at[idx_v.at[pl.ds(off, sz)]],
                    rows_v.at[pl.ds(off, sz)],
                    gsem,
                ).wait()

        pltpu.sync_copy(x_hbm.at[base_e], idx0)
        fire_gathers(idx0, rows0)
        pltpu.async_copy(x_hbm.at[base_e + 1], idx1, isem)

        bufs = ((idx0, rows0, st0), (idx1, rows1, st1))

        def accumulate(rows_c):
            zero = tuple(
                jnp.zeros((_LANES,), jnp.float32) for _ in range(2 * KV))

            def acc_body(j, acc):
                res = list(acc)
                for p in range(2):
                    for k in range(KV):
                        va = rows_c[j + 2 * p,
                                    pl.ds(2 * _LANES * k, 2 * _LANES)]
                        vb = rows_c[j + 2 * p + 1,
                                    pl.ds(2 * _LANES * k, 2 * _LANES)]
                        a, b = plsc.unpack(
                            va + vb, format=plsc.PackFormat.INTERLEAVED)
                        res[2 * k] = res[2 * k] + a
                        res[2 * k + 1] = res[2 * k + 1] + b
                return tuple(res)

            return plsc.parallel_loop(0, S, step=4, carry=zero)(acc_body)

        def body(i, carry):
            for b in (0, 1):
                ec = 2 * i + b
                idx_c, rows_c, st_c = bufs[b]
                idx_n, rows_n, _ = bufs[1 - b]

                @pl.when(ec + 1 < BW)
                def _fire():
                    pltpu.make_async_copy(
                        x_hbm.at[base_e], idx_n, isem).wait()
                    fire_gathers(idx_n, rows_n)

                wait_gathers(idx_c, rows_c)

                @pl.when(ec + 2 < BW)
                def _prefetch():
                    pltpu.async_copy(x_hbm.at[base_e + ec + 2], idx_c, isem)

                acc = accumulate(rows_c)

                @pl.when(ec >= 2)
                def _reclaim():
                    pltpu.make_async_copy(
                        st_c, out_hbm.at[base_e], osem).wait()
                for k in range(KV):
                    st_c[pl.ds(2 * _LANES * k, _LANES)] = acc[2 * k]
                    st_c[pl.ds(2 * _LANES * k + _LANES, _LANES)] = (
                        acc[2 * k + 1])
                pltpu.async_copy(st_c, out_hbm.at[base_e + ec], osem)
            return carry

        lax.fori_loop(0, BW // 2, body, 0)
        pltpu.make_async_copy(st0, out_hbm.at[base_e], osem).wait()
        pltpu.make_async_copy(st1, out_hbm.at[base_e], osem).wait()

    return gather_sum(x, table_b)


def _stats_body(s_ref, w1_ref, b1_ref, o_ref, acc_ref, *, seq):
    i = pl.program_id(0)
    h = (
        jnp.dot(s_ref[...], w1_ref[...], preferred_element_type=jnp.float32)
        * (1.0 / seq)
        + b1_ref[...]
    )
    ps = jnp.sum(h, axis=0, keepdims=True)
    ps2 = jnp.sum(h * h, axis=0, keepdims=True)
    st = jnp.concatenate([ps, ps2], axis=0)

    @pl.when(i == 0)
    def _init():
        acc_ref[...] = st

    @pl.when(i > 0)
    def _acc():
        acc_ref[...] = acc_ref[...] + st

    @pl.when(i == pl.num_programs(0) - 1)
    def _out():
        o_ref[...] = acc_ref[...]


def _apply_body(
    s_ref, w1_ref, b1_ref, st_ref, g_ref, be_ref, w2_ref, b2_ref, o_ref,
    *, seq, batch
):
    h = (
        jnp.dot(s_ref[...], w1_ref[...], preferred_element_type=jnp.float32)
        * (1.0 / seq)
        + b1_ref[...]
    )
    mu = st_ref[0:1, :] * (1.0 / batch)
    var = st_ref[1:2, :] * (1.0 / batch) - mu * mu
    inv = lax.rsqrt(var + 1e-5)
    hn = (h - mu) * (g_ref[...] * inv) + be_ref[...]
    hr = jnp.maximum(hn, 0.0)
    o_ref[...] = (
        jnp.dot(hr, w2_ref[...], preferred_element_type=jnp.float32)
        + b2_ref[...]
    )


def kernel(x, table, W1, b1, gamma, beta, W2, b2):
    B, S = x.shape
    V, D = table.shape
    H = W1.shape[1]
    Lb = W2.shape[1]
    Dp = ((D + 2 * _LANES - 1) // (2 * _LANES)) * (2 * _LANES)

    table_b = _prep_table(table, Dp)
    s = _gather_sum_sc(x, table_b)

    q = np.arange(Dp)
    k32, t = q // 32, q % 32
    perm = np.where(t < 16, 32 * k32 + 2 * t, 32 * k32 + 2 * (t - 16) + 1)
    W1p = jnp.pad(W1, ((0, Dp - D), (0, 0)))[perm, :]
    b1r = b1.reshape(1, H)

    BT = 2048
    grid = (B // BT,)
    sums = pl.pallas_call(
        functools.partial(_stats_body, seq=S),
        grid=grid,
        in_specs=[
            pl.BlockSpec((BT, Dp), lambda i: (i, 0)),
            pl.BlockSpec((Dp, H), lambda i: (0, 0)),
            pl.BlockSpec((1, H), lambda i: (0, 0)),
        ],
        out_specs=pl.BlockSpec((2, H), lambda i: (0, 0)),
        out_shape=jax.ShapeDtypeStruct((2, H), jnp.float32),
        scratch_shapes=[pltpu.VMEM((2, H), jnp.float32)],
    )(s, W1p, b1r)

    out = pl.pallas_call(
        functools.partial(_apply_body, seq=S, batch=B),
        grid=grid,
        in_specs=[
            pl.BlockSpec((BT, Dp), lambda i: (i, 0)),
            pl.BlockSpec((Dp, H), lambda i: (0, 0)),
            pl.BlockSpec((1, H), lambda i: (0, 0)),
            pl.BlockSpec((2, H), lambda i: (0, 0)),
            pl.BlockSpec((1, H), lambda i: (0, 0)),
            pl.BlockSpec((1, H), lambda i: (0, 0)),
            pl.BlockSpec((H, Lb), lambda i: (0, 0)),
            pl.BlockSpec((1, Lb), lambda i: (0, 0)),
        ],
        out_specs=pl.BlockSpec((BT, Lb), lambda i: (i, 0)),
        out_shape=jax.ShapeDtypeStruct((B, Lb), jnp.float32),
    )(s, W1p, b1r, sums, gamma.reshape(1, H), beta.reshape(1, H), W2,
      b2.reshape(1, Lb))
    return out
